# baseline (speedup 1.0000x reference)
import functools

import jax
import jax.numpy as jnp
from jax import lax
from jax.experimental import pallas as pl
from jax.experimental.pallas import tpu as pltpu
from jax.experimental.pallas import tpu_sc as plsc

_NUM_LAYERS = 24
_STM = 2048
_D = 1024
_NC = 2
_NS = 16
_NW = _NC * _NS
_RPW = _STM // _NW
_L = 16

_mesh = plsc.VectorSubcoreMesh(core_axis_name="c", subcore_axis_name="s")


@functools.partial(
    pl.kernel,
    mesh=_mesh,
    out_type=jax.ShapeDtypeStruct((_STM, _D), jnp.float32),
    scratch_types=[
        pltpu.VMEM((_L,), jnp.int32),
        pltpu.VMEM((_RPW,), jnp.int32),
        pltpu.VMEM((_RPW, _D), jnp.float32),
        pltpu.SemaphoreType.DMA,
    ],
)
def _stm_lookup(mem_hbm, layer_hbm, out_hbm, lbuf, idx_v, rows_v, sem):
    wid = lax.axis_index("s") * _NC + lax.axis_index("c")
    base = wid * _RPW
    pltpu.sync_copy(layer_hbm, lbuf)
    row0 = lbuf[...] * _STM + base
    for j in range(_RPW // _L):
        idx_v[pl.ds(j * _L, _L)] = row0 + j * _L + lax.iota(jnp.int32, _L)
    pltpu.async_copy(mem_hbm.at[idx_v], rows_v, sem).wait()
    pltpu.sync_copy(rows_v, out_hbm.at[pl.ds(base, _RPW)])


def kernel(memory, layer):
    mem2d = memory.reshape(_NUM_LAYERS * _STM, _D)
    layer_vec = jnp.full((_L,), layer, dtype=jnp.int32)
    out = _stm_lookup(mem2d, layer_vec)
    return out[None]


# speedup vs baseline: 1.0011x; 1.0011x over previous
import functools

import jax
import jax.numpy as jnp
from jax import lax
from jax.experimental import pallas as pl
from jax.experimental.pallas import tpu as pltpu
from jax.experimental.pallas import tpu_sc as plsc

_NUM_LAYERS = 24
_STM = 2048
_D = 1024
_NC = 2
_NS = 16
_NW = _NC * _NS
_NCH = 8
_CR = 8
_CPL = _STM // _CR
_L = 16

_mesh = plsc.VectorSubcoreMesh(core_axis_name="c", subcore_axis_name="s")


@functools.partial(
    pl.kernel,
    mesh=_mesh,
    out_type=jax.ShapeDtypeStruct((_CPL, _CR, _D), jnp.float32),
    scratch_types=[
        pltpu.VMEM((_L,), jnp.int32),
        pltpu.VMEM((_NCH * 8,), jnp.int32),
        pltpu.VMEM((_NCH, _CR, _D), jnp.float32),
        [pltpu.SemaphoreType.DMA] * _NCH,
        pltpu.SemaphoreType.DMA,
    ],
)
def _stm_lookup(mem_hbm, layer_hbm, out_hbm, lbuf, idx_v, rows_v, gsems, ssem):
    wid = lax.axis_index("s") * _NC + lax.axis_index("c")
    cbase = wid * _NCH
    pltpu.sync_copy(layer_hbm, lbuf)
    half = lax.shift_right_logical(lax.iota(jnp.int32, _L), 3)
    for k in range(_NCH // 2):
        idx_v[pl.ds(k * _L, _L)] = lbuf[...] * _CPL + (cbase + 2 * k) + half
    gets = []
    for j in range(_NCH):
        c = pltpu.async_copy(
            mem_hbm.at[idx_v.at[pl.ds(8 * j, 1)]],
            rows_v.at[pl.ds(j, 1)],
            gsems[j],
        )
        gets.append(c)
    puts = []
    for j in range(_NCH):
        gets[j].wait()
        c = pltpu.async_copy(
            rows_v.at[pl.ds(j, 1)],
            out_hbm.at[pl.ds(cbase + j, 1)],
            ssem,
        )
        puts.append(c)
    for c in puts:
        c.wait()


def kernel(memory, layer):
    mem3d = memory.reshape(_NUM_LAYERS * _CPL, _CR, _D)
    layer_vec = jnp.full((_L,), layer, dtype=jnp.int32)
    out = _stm_lookup(mem3d, layer_vec)
    return out.reshape(1, _STM, _D)
